# Initial kernel scaffold; baseline (speedup 1.0000x reference)
import jax
import jax.numpy as jnp
from jax.experimental import pallas as pl
from jax.experimental.pallas import tpu as pltpu
from jax.experimental.pallas import tpu_sc as plsc

VOCAB = 1000
WINDOW = 64


def _make_gather(num_indices: int, d: int):
  mesh = plsc.VectorSubcoreMesh(core_axis_name="core",
                                subcore_axis_name="subcore")

  @jax.jit
  def gather(table, idx):
    idx2d = idx.reshape((1, num_indices))

    @pl.kernel(
        out_type=jax.ShapeDtypeStruct((num_indices, d), table.dtype),
        mesh=mesh,
    )
    def kernel(x_hbm, i_hbm, o_hbm):
      def body(i_vmem, o_vmem):
        pltpu.sync_copy(x_hbm.at[i_vmem.at[0]], o_vmem)

      pltpu.emit_pipeline(
          body,
          grid=(num_indices // WINDOW,),
          in_specs=[pl.BlockSpec((1, WINDOW), index_map=lambda i: (0, i))],
          out_specs=[pl.BlockSpec((WINDOW, d), index_map=lambda i: (i, 0))],
          core_axis_name=("core", "subcore"),
          dimension_semantics=(pltpu.PARALLEL,),
      )(i_hbm, o_hbm)

    return kernel(table, idx2d)

  return gather


_gather = _make_gather(1024 * 50, VOCAB)


def kernel(x, embedding):
  b, t = x.shape
  flat = x.reshape((b * t,)).astype(jnp.int32)
  out = _gather(embedding, flat)
  return out.reshape((b, t, embedding.shape[1]))


# speedup vs baseline: 1.0194x; 1.0194x over previous
import functools

import jax
import jax.numpy as jnp
from jax import lax
from jax.experimental import pallas as pl
from jax.experimental.pallas import tpu as pltpu
from jax.experimental.pallas import tpu_sc as plsc

VOCAB = 1000
NUM_WORKERS = 32
CHUNK = 80


def _make_gather(num_indices: int, d: int):
  per_worker = num_indices // NUM_WORKERS
  nchunks = per_worker // CHUNK
  mesh = plsc.VectorSubcoreMesh(core_axis_name="core",
                                subcore_axis_name="subcore")

  @jax.jit
  def gather(table, idx):
    @functools.partial(
        pl.kernel,
        out_type=jax.ShapeDtypeStruct((num_indices, d), table.dtype),
        mesh=mesh,
        scratch_types=[
            pltpu.VMEM((per_worker,), jnp.int32),
            pltpu.VMEM((CHUNK, d), jnp.float32),
            pltpu.SemaphoreType.DMA,
        ],
        compiler_params=pltpu.CompilerParams(use_tc_tiling_on_sc=False),
    )
    def kernel(table_hbm, idx_hbm, out_hbm, idx_v, rows_v, sem):
      wid = lax.axis_index("subcore") * 2 + lax.axis_index("core")
      base = wid * per_worker
      pltpu.sync_copy(idx_hbm.at[pl.ds(base, per_worker)], idx_v)

      @pl.loop(0, nchunks)
      def _(j):
        off = j * CHUNK
        pltpu.async_copy(
            table_hbm.at[idx_v.at[pl.ds(off, CHUNK)]], rows_v, sem).wait()
        pltpu.sync_copy(rows_v, out_hbm.at[pl.ds(base + off, CHUNK)])

    return kernel(table, idx)

  return gather


_gather = _make_gather(1024 * 50, VOCAB)


def kernel(x, embedding):
  b, t = x.shape
  flat = x.reshape((b * t,)).astype(jnp.int32)
  out = _gather(embedding, flat)
  return out.reshape((b, t, embedding.shape[1]))


# speedup vs baseline: 1.0274x; 1.0079x over previous
import functools

import jax
import jax.numpy as jnp
from jax import lax
from jax.experimental import pallas as pl
from jax.experimental.pallas import tpu as pltpu
from jax.experimental.pallas import tpu_sc as plsc

VOCAB = 1000
NUM_WORKERS = 32
CHUNK = 40


def _make_gather(num_indices: int, d: int):
  per_worker = num_indices // NUM_WORKERS
  nchunks = per_worker // CHUNK
  npairs = nchunks // 2
  mesh = plsc.VectorSubcoreMesh(core_axis_name="core",
                                subcore_axis_name="subcore")

  @jax.jit
  def gather(table, idx):
    @functools.partial(
        pl.kernel,
        out_type=jax.ShapeDtypeStruct((num_indices, d), table.dtype),
        mesh=mesh,
        scratch_types=[
            pltpu.VMEM((per_worker,), jnp.int32),
            pltpu.VMEM((2, CHUNK, d), jnp.float32),
            pltpu.SemaphoreType.DMA,
            pltpu.SemaphoreType.DMA,
            pltpu.SemaphoreType.DMA,
            pltpu.SemaphoreType.DMA,
        ],
        compiler_params=pltpu.CompilerParams(use_tc_tiling_on_sc=False),
    )
    def kernel(table_hbm, idx_hbm, out_hbm, idx_v, rows_v, g0, g1, w0, w1):
      wid = lax.axis_index("subcore") * 2 + lax.axis_index("core")
      base = wid * per_worker
      pltpu.sync_copy(idx_hbm.at[pl.ds(base, per_worker)], idx_v)

      gsem = (g0, g1)
      wsem = (w0, w1)

      def start_gather(j, b):
        pltpu.async_copy(
            table_hbm.at[idx_v.at[pl.ds(j * CHUNK, CHUNK)]],
            rows_v.at[b], gsem[b])

      def wait_gather(b):
        pltpu.make_async_copy(
            out_hbm.at[pl.ds(0, CHUNK)], rows_v.at[b], gsem[b]).wait()

      def start_write(j, b):
        pltpu.async_copy(
            rows_v.at[b], out_hbm.at[pl.ds(base + j * CHUNK, CHUNK)], wsem[b])

      def wait_write(b):
        pltpu.make_async_copy(
            rows_v.at[b], out_hbm.at[pl.ds(0, CHUNK)], wsem[b]).wait()

      start_gather(0, 0)

      @pl.loop(0, npairs)
      def _(g):
        j0 = 2 * g
        j1 = j0 + 1
        wait_gather(0)
        start_write(j0, 0)

        @pl.when(g > 0)
        def _():
          wait_write(1)

        start_gather(j1, 1)
        wait_gather(1)
        start_write(j1, 1)

        @pl.when(g < npairs - 1)
        def _():
          wait_write(0)
          start_gather(j0 + 2, 0)

      wait_write(0)
      wait_write(1)

    return kernel(table, idx)

  return gather


_gather = _make_gather(1024 * 50, VOCAB)


def kernel(x, embedding):
  b, t = x.shape
  flat = x.reshape((b * t,)).astype(jnp.int32)
  out = _gather(embedding, flat)
  return out.reshape((b, t, embedding.shape[1]))


# speedup vs baseline: 1.4347x; 1.3964x over previous
import functools

import jax
import jax.numpy as jnp
from jax import lax
from jax.experimental import pallas as pl
from jax.experimental.pallas import tpu as pltpu
from jax.experimental.pallas import tpu_sc as plsc

VOCAB = 1000
NUM_WORKERS = 32
KROWS = 32
LANES = 16


def _make_gather(bsz: int, t_len: int, d: int):
  dpad = NUM_WORKERS * KROWS
  nj = bsz // LANES
  npairs = t_len // 2
  mesh = plsc.VectorSubcoreMesh(core_axis_name="core",
                                subcore_axis_name="subcore")

  @jax.jit
  def gather(tpad, xt_flat):

    @functools.partial(
        pl.kernel,
        out_type=jax.ShapeDtypeStruct((t_len, d, bsz), jnp.float32),
        mesh=mesh,
        scratch_types=[
            pltpu.VMEM((KROWS, bsz), jnp.float32),
            pltpu.VMEM((2, bsz), jnp.int32),
            pltpu.VMEM((2, KROWS, bsz), jnp.float32),
            pltpu.SemaphoreType.DMA,
            pltpu.SemaphoreType.DMA,
            pltpu.SemaphoreType.DMA,
            pltpu.SemaphoreType.DMA,
        ],
        compiler_params=pltpu.CompilerParams(needs_layout_passes=False),
    )
    def kernel(tpad_hbm, idx_hbm, p_hbm, tk, idx_v, ob, i0, i1, w0, w1):
      wid = lax.axis_index("subcore") * 2 + lax.axis_index("core")
      ks = wid * KROWS
      last = jnp.equal(wid, NUM_WORKERS - 1)
      isem = (i0, i1)
      wsem = (w0, w1)

      def start_idx(t, p):
        pltpu.async_copy(idx_hbm.at[pl.ds(t * bsz, bsz)], idx_v.at[p],
                         isem[p])

      def wait_idx(p):
        pltpu.make_async_copy(idx_hbm.at[pl.ds(0, bsz)], idx_v.at[p],
                              isem[p]).wait()

      def start_write(t, p):
        @pl.when(jnp.logical_not(last))
        def _():
          pltpu.async_copy(ob.at[p], p_hbm.at[t, pl.ds(ks, KROWS)], wsem[p])

        @pl.when(last)
        def _():
          pltpu.async_copy(ob.at[p, pl.ds(0, d - ks)],
                           p_hbm.at[t, pl.ds(ks, d - ks)], wsem[p])

      def wait_write(p):
        @pl.when(jnp.logical_not(last))
        def _():
          pltpu.make_async_copy(ob.at[p], p_hbm.at[0, pl.ds(0, KROWS)],
                                wsem[p]).wait()

        @pl.when(last)
        def _():
          pltpu.make_async_copy(ob.at[p, pl.ds(0, d - ks)],
                                p_hbm.at[0, pl.ds(0, d - ks)], wsem[p]).wait()

      pltpu.sync_copy(tpad_hbm.at[pl.ds(ks, KROWS)], tk)
      start_idx(0, 0)
      start_idx(1, 1)

      def do_t(g, t, p):
        wait_idx(p)

        @pl.when(g > 0)
        def _():
          wait_write(p)

        @pl.loop(0, nj)
        def _(j):
          vidx = idx_v[p, pl.ds(j * LANES, LANES)]
          for k in range(KROWS):
            krow = jnp.full((LANES,), k, dtype=jnp.int32)
            ob[p, k, pl.ds(j * LANES, LANES)] = plsc.load_gather(
                tk, [krow, vidx])

        start_write(t, p)

        @pl.when(g < npairs - 1)
        def _():
          start_idx(t + 2, p)

      @pl.loop(0, npairs)
      def _(g):
        do_t(g, 2 * g, 0)
        do_t(g, 2 * g + 1, 1)

      wait_write(0)
      wait_write(1)

    return kernel(tpad, xt_flat)

  return gather


_B, _T = 1024, 50
_gather = _make_gather(_B, _T, VOCAB)


def kernel(x, embedding):
  dpad = NUM_WORKERS * KROWS
  tpad = jnp.pad(embedding.astype(jnp.float32).T,
                 ((0, dpad - VOCAB), (0, _B - VOCAB)))
  xt_flat = x.astype(jnp.int32).T.reshape((_T * _B,))
  p = _gather(tpad, xt_flat)
  return jnp.transpose(p, (2, 0, 1))


# speedup vs baseline: 1.4741x; 1.0275x over previous
import functools

import jax
import jax.numpy as jnp
from jax import lax
from jax.experimental import pallas as pl
from jax.experimental.pallas import tpu as pltpu
from jax.experimental.pallas import tpu_sc as plsc

VOCAB = 1000
NUM_WORKERS = 32
KROWS = 32
LANES = 16
JUNROLL = 8


def _make_gather(bsz: int, t_len: int, d: int):
  dpad = NUM_WORKERS * KROWS
  nj = bsz // LANES
  npairs = t_len // 2
  mesh = plsc.VectorSubcoreMesh(core_axis_name="core",
                                subcore_axis_name="subcore")

  @jax.jit
  def gather(tpad, xt_flat):

    @functools.partial(
        pl.kernel,
        out_type=jax.ShapeDtypeStruct((t_len, d, bsz), jnp.float32),
        mesh=mesh,
        scratch_types=[
            pltpu.VMEM((KROWS, bsz), jnp.float32),
            pltpu.VMEM((2, bsz), jnp.int32),
            pltpu.VMEM((2, KROWS, bsz), jnp.float32),
            pltpu.SemaphoreType.DMA,
            pltpu.SemaphoreType.DMA,
            pltpu.SemaphoreType.DMA,
            pltpu.SemaphoreType.DMA,
        ],
        compiler_params=pltpu.CompilerParams(needs_layout_passes=False),
    )
    def kernel(tpad_hbm, idx_hbm, p_hbm, tk, idx_v, ob, i0, i1, w0, w1):
      wid = lax.axis_index("subcore") * 2 + lax.axis_index("core")
      ks = wid * KROWS
      last = jnp.equal(wid, NUM_WORKERS - 1)
      isem = (i0, i1)
      wsem = (w0, w1)

      def start_idx(t, p):
        pltpu.async_copy(idx_hbm.at[pl.ds(t * bsz, bsz)], idx_v.at[p],
                         isem[p])

      def wait_idx(p):
        pltpu.make_async_copy(idx_hbm.at[pl.ds(0, bsz)], idx_v.at[p],
                              isem[p]).wait()

      def start_write(t, p):
        @pl.when(jnp.logical_not(last))
        def _():
          pltpu.async_copy(ob.at[p], p_hbm.at[t, pl.ds(ks, KROWS)], wsem[p])

        @pl.when(last)
        def _():
          pltpu.async_copy(ob.at[p, pl.ds(0, d - ks)],
                           p_hbm.at[t, pl.ds(ks, d - ks)], wsem[p])

      def wait_write(p):
        @pl.when(jnp.logical_not(last))
        def _():
          pltpu.make_async_copy(ob.at[p], p_hbm.at[0, pl.ds(0, KROWS)],
                                wsem[p]).wait()

        @pl.when(last)
        def _():
          pltpu.make_async_copy(ob.at[p, pl.ds(0, d - ks)],
                                p_hbm.at[0, pl.ds(0, d - ks)], wsem[p]).wait()

      pltpu.sync_copy(tpad_hbm.at[pl.ds(ks, KROWS)], tk)
      start_idx(0, 0)
      start_idx(1, 1)

      def do_t(g, t, p):
        wait_idx(p)

        @pl.when(g > 0)
        def _():
          wait_write(p)

        @pl.loop(0, nj // JUNROLL)
        def _(jj):
          base = jj * (LANES * JUNROLL)
          vidx = [idx_v[p, pl.ds(base + u * LANES, LANES)]
                  for u in range(JUNROLL)]
          for k in range(KROWS):
            krow = jnp.full((LANES,), k, dtype=jnp.int32)
            for u in range(JUNROLL):
              ob[p, k, pl.ds(base + u * LANES, LANES)] = plsc.load_gather(
                  tk, [krow, vidx[u]])

        start_write(t, p)

        @pl.when(g < npairs - 1)
        def _():
          start_idx(t + 2, p)

      @pl.loop(0, npairs)
      def _(g):
        do_t(g, 2 * g, 0)
        do_t(g, 2 * g + 1, 1)

      wait_write(0)
      wait_write(1)

    return kernel(tpad, xt_flat)

  return gather


_B, _T = 1024, 50
_gather = _make_gather(_B, _T, VOCAB)


def kernel(x, embedding):
  dpad = NUM_WORKERS * KROWS
  tpad = jnp.pad(embedding.astype(jnp.float32).T,
                 ((0, dpad - VOCAB), (0, _B - VOCAB)))
  xt_flat = x.astype(jnp.int32).T.reshape((_T * _B,))
  p = _gather(tpad, xt_flat)
  return jnp.transpose(p, (2, 0, 1))


# speedup vs baseline: 4.1491x; 2.8146x over previous
import functools

import jax
import jax.numpy as jnp
from jax import lax
from jax.experimental import pallas as pl
from jax.experimental.pallas import tpu as pltpu
from jax.experimental.pallas import tpu_sc as plsc

VOCAB = 1000
NUM_WORKERS = 32
KROWS = 32
LANES = 16
JUNROLL = 8


def _make_gather(bsz: int, t_len: int, d: int):
  dpad = NUM_WORKERS * KROWS
  nj = bsz // LANES
  npairs = t_len // 2
  mesh = plsc.VectorSubcoreMesh(core_axis_name="core",
                                subcore_axis_name="subcore")

  @jax.jit
  def gather(tpad, xt_flat):

    @functools.partial(
        pl.kernel,
        out_type=jax.ShapeDtypeStruct((t_len, d, bsz), jnp.float32),
        mesh=mesh,
        scratch_types=[
            pltpu.VMEM((KROWS, bsz), jnp.float32),
            pltpu.VMEM((2, bsz), jnp.int32),
            pltpu.VMEM((2, KROWS, bsz), jnp.float32),
            pltpu.SemaphoreType.DMA,
            pltpu.SemaphoreType.DMA,
            pltpu.SemaphoreType.DMA,
            pltpu.SemaphoreType.DMA,
        ],
        compiler_params=pltpu.CompilerParams(needs_layout_passes=False),
    )
    def kernel(tpad_hbm, idx_hbm, p_hbm, tk, idx_v, ob, i0, i1, w0, w1):
      wid = lax.axis_index("subcore") * 2 + lax.axis_index("core")
      ks = wid * KROWS
      last = jnp.equal(wid, NUM_WORKERS - 1)
      isem = (i0, i1)
      wsem = (w0, w1)

      def start_idx(t, p):
        pltpu.async_copy(idx_hbm.at[pl.ds(t * bsz, bsz)], idx_v.at[p],
                         isem[p])

      def wait_idx(p):
        pltpu.make_async_copy(idx_hbm.at[pl.ds(0, bsz)], idx_v.at[p],
                              isem[p]).wait()

      def start_write(t, p):
        @pl.when(jnp.logical_not(last))
        def _():
          pltpu.async_copy(ob.at[p], p_hbm.at[t, pl.ds(ks, KROWS)], wsem[p])

        @pl.when(last)
        def _():
          pltpu.async_copy(ob.at[p, pl.ds(0, d - ks)],
                           p_hbm.at[t, pl.ds(ks, d - ks)], wsem[p])

      def wait_write(p):
        @pl.when(jnp.logical_not(last))
        def _():
          pltpu.make_async_copy(ob.at[p], p_hbm.at[0, pl.ds(0, KROWS)],
                                wsem[p]).wait()

        @pl.when(last)
        def _():
          pltpu.make_async_copy(ob.at[p, pl.ds(0, d - ks)],
                                p_hbm.at[0, pl.ds(0, d - ks)], wsem[p]).wait()

      pltpu.sync_copy(tpad_hbm.at[pl.ds(ks, KROWS)], tk)
      start_idx(0, 0)
      start_idx(1, 1)

      def do_t(g, t, p):
        wait_idx(p)

        @pl.when(g > 0)
        def _():
          wait_write(p)

        @pl.loop(0, nj // JUNROLL)
        def _(jj):
          base = jj * (LANES * JUNROLL)
          vidx = [idx_v[p, pl.ds(base + u * LANES, LANES)]
                  for u in range(JUNROLL)]
          for k in range(KROWS):
            krow = jnp.full((LANES,), k, dtype=jnp.int32)
            vals = [plsc.load_gather(tk, [krow, vidx[u]])
                    for u in range(JUNROLL)]
            for u in range(JUNROLL):
              ob[p, k, pl.ds(base + u * LANES, LANES)] = vals[u]

        start_write(t, p)

        @pl.when(g < npairs - 1)
        def _():
          start_idx(t + 2, p)

      @pl.loop(0, npairs)
      def _(g):
        do_t(g, 2 * g, 0)
        do_t(g, 2 * g + 1, 1)

      wait_write(0)
      wait_write(1)

    return kernel(tpad, xt_flat)

  return gather


_B, _T = 1024, 50
_gather = _make_gather(_B, _T, VOCAB)


def kernel(x, embedding):
  dpad = NUM_WORKERS * KROWS
  tpad = jnp.pad(embedding.astype(jnp.float32).T,
                 ((0, dpad - VOCAB), (0, _B - VOCAB)))
  xt_flat = x.astype(jnp.int32).T.reshape((_T * _B,))
  p = _gather(tpad, xt_flat)
  return jnp.transpose(p, (2, 0, 1))
